# Initial kernel scaffold; baseline (speedup 1.0000x reference)
import jax
import jax.numpy as jnp
from jax.experimental import pallas as pl


def kernel(cond_obs, cond_mask, side_info, noisy_data, diffusion_step, proj1_w, proj1_b, proj2_w, proj2_b, fwd_Wr, fwd_br, fwd_Wz, fwd_bz, fwd_Wc, fwd_bc, fwd_Wp, fwd_bp, bwd_Wr, bwd_br, bwd_Wz, bwd_bz, bwd_Wc, bwd_bc, bwd_Wp, bwd_bp, mlp_w1, mlp_b1, mlp_w2, mlp_b2, out_w, out_b):
    raise NotImplementedError("write your pallas kernel here")


# speedup vs baseline: 1.8082x; 1.8082x over previous
import numpy as np
import jax
import jax.numpy as jnp
from jax.experimental import pallas as pl
from jax.experimental.pallas import tpu as pltpu

NUM_STEPS = 50
EMB_DIM = 64
K = 207
H = 32
B = 8
L = 64

_IDX_S = np.array([1] + list(range(2, 18)) + [82], dtype=np.int32)
_IDX_GS = _IDX_S + 115
_IDX_HX = np.array(list(range(83, 115)) + [0], dtype=np.int32)
_IDX_GHX = _IDX_HX + 115


def _emb_table():
    half = EMB_DIM // 2
    steps = np.arange(NUM_STEPS, dtype=np.float64)[:, None]
    freqs = (10.0 ** (np.arange(half, dtype=np.float64) / (half - 1) * 4.0))[None, :]
    t = steps * freqs
    return np.concatenate([np.sin(t), np.cos(t)], axis=1).astype(np.float32)


def _adj_t():
    i = np.arange(K, dtype=np.float64)
    adj = np.exp(-np.square(i[:, None] - i[None, :]) / 2.0) - np.eye(K)
    adj = adj / adj.sum(axis=1, keepdims=True)
    return np.ascontiguousarray(adj.T).astype(np.float32)


def _kern(dstep_ref, s_ref, at_ref, table_ref,
          p1_ref, p1b_ref, p2_ref, p2b_ref,
          wpf_ref, wpb_ref,
          wrzhxf_ref, wrzgf_ref, wchxf_ref, wcgf_ref,
          wrzhxb_ref, wrzgb_ref, wchxb_ref, wcgb_ref,
          ws_ref, wg_ref, wemb_ref,
          w1_ref, w1mx_ref, w2_ref, ow_ref, owmx_ref,
          y_ref, imp_ref,
          contrib, fh, fs, bh, bs):
    b = pl.program_id(0)
    at = at_ref[...]
    ones_row = jnp.ones((1, K), jnp.float32)

    step = dstep_ref[b]
    emb = table_ref[pl.ds(step, 1), :]
    e = emb @ p1_ref[...] + p1b_ref[...]
    e = e * jax.nn.sigmoid(e)
    e = e @ p2_ref[...] + p2b_ref[...]
    e = e * jax.nn.sigmoid(e)
    ebias = jax.lax.dot_general(wemb_ref[...], e, (((1,), (1,)), ((), ())))
    eb_full = jnp.dot(ebias, ones_row)

    ws = ws_ref[...]
    wg = wg_ref[...]

    def pre_body(t, _):
        s20 = s_ref[0, t]
        g20 = jnp.dot(s20, at)
        contrib[t] = jnp.dot(ws, s20) + jnp.dot(wg, g20) + eb_full
        return 0

    jax.lax.fori_loop(0, L, pre_body, 0, unroll=False)

    def run_scan(wpt, wrzhx, wrzg, wchx, wcg, c_off, h_scr, s_scr, rev):
        def body(i, h):
            t = (L - 1 - i) if rev else i
            m_row = s_ref[0, t, 0:1, :][0]
            x_row = s_ref[0, t, 1:2, :][0]
            h1 = jnp.concatenate([h, ones_row], axis=0)
            pred = jnp.dot(wpt, h1)
            xf = m_row * x_row + (1.0 - m_row) * pred
            hx = jnp.concatenate([h, xf], axis=0)
            g = jnp.dot(hx, at)
            ct = contrib[t, c_off:c_off + 96]
            rz = ct[0:64] + jnp.dot(wrzhx, hx) + jnp.dot(wrzg, g)
            r = jax.nn.sigmoid(rz[0:32])
            z = jax.nn.sigmoid(rz[32:64])
            rhx = jnp.concatenate([r * h, xf], axis=0)
            g2 = jnp.dot(rhx, at)
            c = jnp.tanh(ct[64:96] + jnp.dot(wchx, rhx) + jnp.dot(wcg, g2))
            h_new = z * h + (1.0 - z) * c
            h_scr[t] = h_new
            s_scr[t] = jnp.dot(h_new, at)
            return h_new

        jax.lax.fori_loop(0, L, body, jnp.zeros((H, K), jnp.float32),
                          unroll=False)

    run_scan(wpf_ref[...], wrzhxf_ref[...], wrzgf_ref[...],
             wchxf_ref[...], wcgf_ref[...], 0, fh, fs, False)
    run_scan(wpb_ref[...], wrzhxb_ref[...], wrzgb_ref[...],
             wchxb_ref[...], wcgb_ref[...], 96, bh, bs, True)

    w1 = w1_ref[...]
    w1mx = w1mx_ref[...]
    w2 = w2_ref[...]
    ow = ow_ref[...]
    owmx = owmx_ref[...]

    def out_body(t, _):
        hcat = jnp.concatenate([fh[t], fs[t], bh[t], bs[t]], axis=0)
        mxo = s_ref[0, t, 0:3, :]
        m_row = mxo[0:1]
        x_row = mxo[1:2]
        y1 = jax.nn.relu(jnp.dot(w1, hcat) + jnp.dot(w1mx, mxo))
        y1e = jnp.concatenate([y1, ones_row], axis=0)
        yhat = jnp.dot(w2, y1e)
        imp = m_row * x_row + (1.0 - m_row) * yhat
        y = jnp.dot(ow, hcat) + jnp.dot(owmx, mxo)
        y_ref[0, pl.ds(t, 1), :] = y
        imp_ref[0, pl.ds(t, 1), :] = imp
        return 0

    jax.lax.fori_loop(0, L, out_body, 0, unroll=False)


def kernel(cond_obs, cond_mask, side_info, noisy_data, diffusion_step,
           proj1_w, proj1_b, proj2_w, proj2_b,
           fwd_Wr, fwd_br, fwd_Wz, fwd_bz, fwd_Wc, fwd_bc, fwd_Wp, fwd_bp,
           bwd_Wr, bwd_br, bwd_Wz, bwd_bz, bwd_Wc, bwd_bc, bwd_Wp, bwd_bp,
           mlp_w1, mlp_b1, mlp_w2, mlp_b2, out_w, out_b):
    f32 = jnp.float32
    x = cond_obs[:, 0].transpose(0, 2, 1)
    m = cond_mask[:, 0].transpose(0, 2, 1)
    v = noisy_data[:, 0].transpose(0, 2, 1)
    ones_ch = jnp.ones((B, L, 1, K), f32)
    s_stat = jnp.concatenate(
        [m[:, :, None, :], x[:, :, None, :], ones_ch,
         side_info.transpose(0, 3, 1, 2), v[:, :, None, :]],
        axis=2)
    dstep = diffusion_step.astype(jnp.int32)

    def gate_prep(Wr, Wz, Wc, br, bz, bc):
        wrz = jnp.concatenate([Wr, Wz], axis=1)
        return dict(
            rz_hx=wrz[_IDX_HX].T, rz_g=wrz[_IDX_GHX].T,
            c_hx=Wc[_IDX_HX].T, c_g=Wc[_IDX_GHX].T,
            rz_s=wrz[_IDX_S].T, rz_gs=wrz[_IDX_GS].T,
            c_s=Wc[_IDX_S].T, c_gs=Wc[_IDX_GS].T,
            rz_emb=(wrz[18:82] + wrz[133:197]).T,
            c_emb=(Wc[18:82] + Wc[133:197]).T,
            brz=jnp.concatenate([br, bz])[:, None],
            bc=bc[:, None],
        )

    gf = gate_prep(fwd_Wr, fwd_Wz, fwd_Wc, fwd_br, fwd_bz, fwd_bc)
    gb = gate_prep(bwd_Wr, bwd_Wz, bwd_Wc, bwd_br, bwd_bz, bwd_bc)

    ws18 = jnp.concatenate([gf["rz_s"], gf["c_s"], gb["rz_s"], gb["c_s"]], 0)
    wg18 = jnp.concatenate([gf["rz_gs"], gf["c_gs"], gb["rz_gs"], gb["c_gs"]], 0)
    wemb_all = jnp.concatenate(
        [gf["rz_emb"], gf["c_emb"], gb["rz_emb"], gb["c_emb"]], 0)
    b_all = jnp.concatenate([gf["brz"], gf["bc"], gb["brz"], gb["bc"]], 0)
    zcol = jnp.zeros((192, 1), f32)
    ws_aug = jnp.concatenate(
        [ws18[:, 0:1], zcol, b_all, ws18[:, 1:17], ws18[:, 17:18]], axis=1)
    wg_aug = jnp.concatenate(
        [wg18[:, 0:1], zcol, zcol, wg18[:, 1:17], wg18[:, 17:18]], axis=1)

    wpf_aug = jnp.concatenate([fwd_Wp.T, fwd_bp.reshape(1, 1)], axis=1)
    wpb_aug = jnp.concatenate([bwd_Wp.T, bwd_bp.reshape(1, 1)], axis=1)
    w1mx = jnp.concatenate(
        [mlp_w1[128:129].T, mlp_w1[129:130].T, mlp_b1[:, None]], axis=1)
    owmx = jnp.concatenate(
        [out_w[128:129].T, out_w[129:130].T, out_b.reshape(1, 1)], axis=1)
    w2_aug = jnp.concatenate([mlp_w2.T, mlp_b2.reshape(1, 1)], axis=1)

    operands = [
        dstep,
        s_stat,
        jnp.asarray(_adj_t()), jnp.asarray(_emb_table()),
        proj1_w, proj1_b[None, :], proj2_w, proj2_b[None, :],
        wpf_aug, wpb_aug,
        gf["rz_hx"], gf["rz_g"], gf["c_hx"], gf["c_g"],
        gb["rz_hx"], gb["rz_g"], gb["c_hx"], gb["c_g"],
        ws_aug, wg_aug, wemb_all,
        mlp_w1[:128].T, w1mx, w2_aug,
        out_w[:128].T, owmx,
    ]

    def batched(shape):
        nd = len(shape)
        return pl.BlockSpec((1,) + shape[1:],
                            lambda b, d, _n=nd: (b,) + (0,) * (_n - 1))

    def full(shape):
        nd = len(shape)
        return pl.BlockSpec(shape, lambda b, d, _n=nd: (0,) * _n)

    in_specs = []
    for op in operands[1:]:
        if op.ndim >= 3 and op.shape[0] == B:
            in_specs.append(batched(op.shape))
        else:
            in_specs.append(full(op.shape))

    grid_spec = pltpu.PrefetchScalarGridSpec(
        num_scalar_prefetch=1,
        grid=(B,),
        in_specs=in_specs,
        out_specs=[batched((B, L, K)), batched((B, L, K))],
        scratch_shapes=[
            pltpu.VMEM((L, 192, K), f32),
            pltpu.VMEM((L, H, K), f32), pltpu.VMEM((L, H, K), f32),
            pltpu.VMEM((L, H, K), f32), pltpu.VMEM((L, H, K), f32),
        ],
    )

    yt, impt = pl.pallas_call(
        _kern,
        grid_spec=grid_spec,
        out_shape=[jax.ShapeDtypeStruct((B, L, K), f32),
                   jax.ShapeDtypeStruct((B, L, K), f32)],
        compiler_params=pltpu.CompilerParams(
            dimension_semantics=("arbitrary",)),
    )(*operands)

    y = yt.transpose(0, 2, 1)[:, None, :, :]
    imp = impt.transpose(0, 2, 1)[:, None, :, :]
    return (y, imp)
